# baseline (speedup 1.0000x reference)
import functools

import jax
import jax.numpy as jnp
from jax import lax
from jax.experimental import pallas as pl
from jax.experimental.pallas import tpu as pltpu
from jax.experimental.pallas import tpu_sc as plsc

N_NODES = 10000
NPAD = 10240
N_EDGES = 320000
NW = 32
CHUNK = 128
NCHUNK = 80
EPAD = CHUNK * NCHUNK * NW
RPT = NPAD // 16


def _mm_kernel(x_ref, w_ref, o_ref):
    o_ref[...] = jnp.dot(x_ref[...], w_ref[...],
                         preferred_element_type=jnp.float32
                         ).astype(jnp.bfloat16)


def _matmul(x, w, block_rows=1000):
    n, k = x.shape
    m = w.shape[1]
    return pl.pallas_call(
        _mm_kernel,
        grid=(n // block_rows,),
        in_specs=[pl.BlockSpec((block_rows, k), lambda i: (i, 0)),
                  pl.BlockSpec((k, m), lambda i: (0, 0))],
        out_specs=pl.BlockSpec((block_rows, m), lambda i: (i, 0)),
        out_shape=jax.ShapeDtypeStruct((n, m), jnp.bfloat16),
    )(x, w)


def _relu_mm_kernel(p_ref, w_ref, o_ref):
    h = jnp.maximum(p_ref[0] + p_ref[1], 0.0)
    o_ref[...] = jnp.dot(h, w_ref[...], preferred_element_type=jnp.float32
                         ).astype(jnp.bfloat16)


def _relu_matmul(p, w, block_rows=640):
    n, k = p.shape[1], p.shape[2]
    m = w.shape[1]
    return pl.pallas_call(
        _relu_mm_kernel,
        grid=(n // block_rows,),
        in_specs=[pl.BlockSpec((2, block_rows, k), lambda i: (0, i, 0)),
                  pl.BlockSpec((k, m), lambda i: (0, 0))],
        out_specs=pl.BlockSpec((block_rows, m), lambda i: (i, 0)),
        out_shape=jax.ShapeDtypeStruct((n, m), jnp.bfloat16),
    )(p, w)


def _addslice_kernel(p_ref, o_ref):
    s = p_ref[0] + p_ref[1]
    o_ref[...] = s[:, :40]


def _add_slice(p, block_rows=1000):
    d = p.shape[2]
    return pl.pallas_call(
        _addslice_kernel,
        grid=(N_NODES // block_rows,),
        in_specs=[pl.BlockSpec((2, block_rows, d), lambda i: (0, i, 0))],
        out_specs=pl.BlockSpec((block_rows, 40), lambda i: (i, 0)),
        out_shape=jax.ShapeDtypeStruct((N_NODES, 40), jnp.float32),
    )(p)


def _spmm_partials(e4, v3, y, d_out, tc_tiling):
    bf16_gather = y.dtype == jnp.bfloat16
    d = y.shape[1]
    mesh = plsc.VectorSubcoreMesh(core_axis_name="c", subcore_axis_name="s")

    @functools.partial(
        pl.kernel,
        mesh=mesh,
        out_type=jax.ShapeDtypeStruct((2, NPAD, d_out), jnp.float32),
        compiler_params=pltpu.CompilerParams(use_tc_tiling_on_sc=tc_tiling,
                                             needs_layout_passes=False),
        scratch_types=[
            pltpu.VMEM((2, 2, CHUNK), jnp.int32),
            pltpu.VMEM((2, CHUNK), jnp.float32),
            pltpu.VMEM((2, CHUNK, d), y.dtype),
            pltpu.VMEM((CHUNK, d_out), jnp.float32),
            pltpu.SemaphoreType.DMA,
            pltpu.SemaphoreType.DMA,
            pltpu.VMEM_SHARED((NPAD, d_out), jnp.float32),
        ],
    )
    def k(e_hbm, v_hbm, y_hbm, out_hbm,
          idx_v, val_v, rows_v, msg_v, sem0, sem1, acc):
        cid = lax.axis_index("c")
        sid = lax.axis_index("s")
        wid = cid * 16 + sid

        zvec = jnp.zeros((16,), jnp.float32)

        def zrow(i, carry):
            for f in range(d_out // 16):
                msg_v[i, pl.ds(16 * f, 16)] = zvec
            return carry

        lax.fori_loop(0, CHUNK, zrow, 0)
        for b in range(RPT // CHUNK):
            pltpu.sync_copy(msg_v,
                            acc.at[pl.ds(sid * RPT + b * CHUNK, CHUNK)])
        plsc.subcore_barrier()

        pltpu.sync_copy(e_hbm.at[wid, 0], idx_v.at[0])
        pltpu.sync_copy(v_hbm.at[wid, 0], val_v.at[0])
        pltpu.async_copy(y_hbm.at[idx_v.at[0, 1]], rows_v.at[0], sem0)

        def process(c, b):
            def egroup(g, ecarry):
                vals = val_v[b, pl.ds(g * 16, 16)]
                for j in range(16):
                    v = vals[j]
                    e = g * 16 + j
                    if bf16_gather:
                        for f in range(d // 32):
                            x32 = rows_v[b, e, pl.ds(32 * f, 32)]
                            lo, hi = plsc.unpack(
                                x32, format=plsc.PackFormat.INTERLEAVED,
                                preferred_element_type=jnp.float32)
                            if 32 * f < d_out:
                                msg_v[e, pl.ds(32 * f, 16)] = lo * v
                            if 32 * f + 16 < d_out:
                                msg_v[e, pl.ds(32 * f + 16, 16)] = hi * v
                    else:
                        for f in range(d // 16):
                            sl = pl.ds(16 * f, 16)
                            msg_v[e, sl] = rows_v[b, e, sl] * v
                return ecarry

            lax.fori_loop(0, CHUNK // 16, egroup, 0)
            pltpu.sync_copy(msg_v, acc.at[idx_v.at[b, 0]], add=True)

        def body(i, carry):
            c0 = 2 * i
            pltpu.sync_copy(e_hbm.at[wid, c0 + 1], idx_v.at[1])
            pltpu.sync_copy(v_hbm.at[wid, c0 + 1], val_v.at[1])
            pltpu.async_copy(y_hbm.at[idx_v.at[1, 1]], rows_v.at[1], sem1)
            pltpu.make_async_copy(y_hbm.at[idx_v.at[0, 1]], rows_v.at[0],
                                  sem0).wait()
            process(c0, 0)

            @pl.when(i < NCHUNK // 2 - 1)
            def _():
                pltpu.sync_copy(e_hbm.at[wid, c0 + 2], idx_v.at[0])
                pltpu.sync_copy(v_hbm.at[wid, c0 + 2], val_v.at[0])
                pltpu.async_copy(y_hbm.at[idx_v.at[0, 1]], rows_v.at[0],
                                 sem0)

            pltpu.make_async_copy(y_hbm.at[idx_v.at[1, 1]], rows_v.at[1],
                                  sem1).wait()
            process(c0 + 1, 1)
            return carry

        lax.fori_loop(0, NCHUNK // 2, body, 0)
        plsc.subcore_barrier()

        pltpu.sync_copy(acc.at[pl.ds(sid * RPT, RPT)],
                        out_hbm.at[cid, pl.ds(sid * RPT, RPT)])

    return k(e4, v3, y)


def _pack_edges(adj_edge_index, adj_values):
    row = adj_edge_index[0].astype(jnp.int32)
    col = adj_edge_index[1].astype(jnp.int32)
    e = jnp.stack([row, col])
    pad = jnp.zeros((2, EPAD - N_EDGES), jnp.int32)
    e = jnp.concatenate([e, pad], axis=1)
    v = jnp.concatenate(
        [adj_values, jnp.zeros((EPAD - N_EDGES,), jnp.float32)])
    e4 = e.reshape(2, NW, NCHUNK, CHUNK).transpose(1, 2, 0, 3)
    v3 = v.reshape(NW, NCHUNK, CHUNK)
    return e4, v3


def _unpack_perm(k):
    idx = []
    for f in range(k // 32):
        idx.extend(range(32 * f, 32 * f + 32, 2))
        idx.extend(range(32 * f + 1, 32 * f + 32, 2))
    return jnp.array(idx, jnp.int32)


def _dest_to_src(j):
    f, r = divmod(j, 32)
    return 32 * f + (2 * r if r < 16 else 2 * (r - 16) + 1)


@jax.jit
def kernel(adj_edge_index, adj_values, x, W1, W2):
    e4, v3 = _pack_edges(adj_edge_index, adj_values)
    cols = jnp.array([_dest_to_src(j) for j in range(W2.shape[1])], jnp.int32)
    w2p = jnp.zeros((W2.shape[0], 64), jnp.float32)
    w2p = w2p.at[:, cols].set(W2)
    w2p = w2p[_unpack_perm(W2.shape[0])]

    y1 = _matmul(x, W1)
    p1 = _spmm_partials(e4, v3, y1, 128, False)
    y2 = _relu_matmul(p1, w2p)
    p2 = _spmm_partials(e4, v3, y2, 48, False)
    return _add_slice(p2)


# speedup vs baseline: 1.0582x; 1.0582x over previous
import functools

import jax
import jax.numpy as jnp
from jax import lax
from jax.experimental import pallas as pl
from jax.experimental.pallas import tpu as pltpu
from jax.experimental.pallas import tpu_sc as plsc

N_NODES = 10000
NPAD = 10240
N_EDGES = 320000
NW = 32
CHUNK = 128
NCHUNK = 80
EPAD = CHUNK * NCHUNK * NW
RPT = NPAD // 16


def _mm_kernel(x_ref, w_ref, o_ref):
    o_ref[...] = jnp.dot(x_ref[...], w_ref[...],
                         preferred_element_type=jnp.float32
                         ).astype(jnp.bfloat16)


def _matmul(x, w, block_rows=1000):
    n, k = x.shape
    m = w.shape[1]
    return pl.pallas_call(
        _mm_kernel,
        grid=(n // block_rows,),
        in_specs=[pl.BlockSpec((block_rows, k), lambda i: (i, 0)),
                  pl.BlockSpec((k, m), lambda i: (0, 0))],
        out_specs=pl.BlockSpec((block_rows, m), lambda i: (i, 0)),
        out_shape=jax.ShapeDtypeStruct((n, m), jnp.bfloat16),
    )(x, w)


def _relu_mm_kernel(p_ref, w_ref, o_ref):
    h = jnp.maximum(p_ref[0] + p_ref[1], 0.0)
    o_ref[...] = jnp.dot(h, w_ref[...], preferred_element_type=jnp.float32)


def _relu_matmul(p, w, block_rows=640):
    n, k = p.shape[1], p.shape[2]
    m = w.shape[1]
    return pl.pallas_call(
        _relu_mm_kernel,
        grid=(n // block_rows,),
        in_specs=[pl.BlockSpec((2, block_rows, k), lambda i: (0, i, 0)),
                  pl.BlockSpec((k, m), lambda i: (0, 0))],
        out_specs=pl.BlockSpec((block_rows, m), lambda i: (i, 0)),
        out_shape=jax.ShapeDtypeStruct((n, m), jnp.float32),
    )(p, w)


def _addslice_kernel(p_ref, o_ref):
    s = p_ref[0] + p_ref[1]
    o_ref[...] = s[:, :40]


def _add_slice(p, block_rows=1000):
    d = p.shape[2]
    return pl.pallas_call(
        _addslice_kernel,
        grid=(N_NODES // block_rows,),
        in_specs=[pl.BlockSpec((2, block_rows, d), lambda i: (0, i, 0))],
        out_specs=pl.BlockSpec((block_rows, 40), lambda i: (i, 0)),
        out_shape=jax.ShapeDtypeStruct((N_NODES, 40), jnp.float32),
    )(p)


def _spmm_partials(e4, v3, y, d, tc_tiling, async_scatter=False):
    bf16_gather = y.dtype == jnp.bfloat16
    mesh = plsc.VectorSubcoreMesh(core_axis_name="c", subcore_axis_name="s")

    @functools.partial(
        pl.kernel,
        mesh=mesh,
        out_type=jax.ShapeDtypeStruct((2, NPAD, d), jnp.float32),
        compiler_params=pltpu.CompilerParams(use_tc_tiling_on_sc=tc_tiling,
                                             needs_layout_passes=False),
        scratch_types=[
            pltpu.VMEM((2, 2, CHUNK), jnp.int32),
            pltpu.VMEM((2, CHUNK), jnp.float32),
            pltpu.VMEM((2, CHUNK), jnp.int32),
            pltpu.VMEM((2, CHUNK, d), y.dtype),
            pltpu.VMEM((2 if async_scatter else 1, CHUNK, d),
                       jnp.float32),
            pltpu.SemaphoreType.DMA,
            pltpu.SemaphoreType.DMA,
            pltpu.SemaphoreType.DMA,
            pltpu.SemaphoreType.DMA,
            pltpu.VMEM_SHARED((NPAD, d), jnp.float32),
        ],
    )
    def k(e_hbm, v_hbm, y_hbm, out_hbm,
          idx_v, val_v, srow_v, rows_v, msg_v, sem0, sem1, ssem0, ssem1,
          acc):
        cid = lax.axis_index("c")
        sid = lax.axis_index("s")
        wid = cid * 16 + sid

        zvec = jnp.zeros((16,), jnp.float32)

        def zrow(i, carry):
            for f in range(d // 16):
                msg_v[0, i, pl.ds(16 * f, 16)] = zvec
            return carry

        lax.fori_loop(0, CHUNK, zrow, 0)
        for b in range(RPT // CHUNK):
            pltpu.sync_copy(msg_v.at[0],
                            acc.at[pl.ds(sid * RPT + b * CHUNK, CHUNK)])
        plsc.subcore_barrier()

        pltpu.sync_copy(e_hbm.at[wid, 0], idx_v.at[0])
        pltpu.sync_copy(v_hbm.at[wid, 0], val_v.at[0])
        pltpu.async_copy(y_hbm.at[idx_v.at[0, 1]], rows_v.at[0], sem0)

        def process(c, b):
            mb = b if async_scatter else 0

            def egroup(g, ecarry):
                vals = val_v[b, pl.ds(g * 16, 16)]
                for j in range(16):
                    v = vals[j]
                    e = g * 16 + j
                    if bf16_gather:
                        for f in range(d // 32):
                            x32 = rows_v[b, e, pl.ds(32 * f, 32)]
                            lo, hi = plsc.unpack(
                                x32, format=plsc.PackFormat.INTERLEAVED,
                                preferred_element_type=jnp.float32)
                            msg_v[mb, e, pl.ds(32 * f, 16)] = lo * v
                            msg_v[mb, e, pl.ds(32 * f + 16, 16)] = hi * v
                    else:
                        for f in range(d // 16):
                            sl = pl.ds(16 * f, 16)
                            msg_v[mb, e, sl] = rows_v[b, e, sl] * v
                return ecarry

            lax.fori_loop(0, CHUNK // 16, egroup, 0)
            if async_scatter:
                ssem = ssem0 if b == 0 else ssem1
                for g in range(CHUNK // 16):
                    srow_v[b, pl.ds(g * 16, 16)] = \
                        idx_v[b, 0, pl.ds(g * 16, 16)]
                pltpu.async_copy(msg_v.at[mb], acc.at[srow_v.at[b]], ssem,
                                 add=True)
            else:
                pltpu.sync_copy(msg_v.at[0], acc.at[idx_v.at[b, 0]],
                                add=True)

        def swait(b):
            pltpu.make_async_copy(msg_v.at[b], acc.at[srow_v.at[b]],
                                  ssem0 if b == 0 else ssem1).wait()

        def body(i, carry):
            c0 = 2 * i
            pltpu.sync_copy(e_hbm.at[wid, c0 + 1], idx_v.at[1])
            pltpu.sync_copy(v_hbm.at[wid, c0 + 1], val_v.at[1])
            pltpu.async_copy(y_hbm.at[idx_v.at[1, 1]], rows_v.at[1], sem1)
            pltpu.make_async_copy(y_hbm.at[idx_v.at[0, 1]], rows_v.at[0],
                                  sem0).wait()
            if async_scatter:
                @pl.when(i > 0)
                def _():
                    swait(0)
            process(c0, 0)

            @pl.when(i < NCHUNK // 2 - 1)
            def _():
                pltpu.sync_copy(e_hbm.at[wid, c0 + 2], idx_v.at[0])
                pltpu.sync_copy(v_hbm.at[wid, c0 + 2], val_v.at[0])
                pltpu.async_copy(y_hbm.at[idx_v.at[0, 1]], rows_v.at[0],
                                 sem0)

            pltpu.make_async_copy(y_hbm.at[idx_v.at[1, 1]], rows_v.at[1],
                                  sem1).wait()
            if async_scatter:
                @pl.when(i > 0)
                def _():
                    swait(1)
            process(c0 + 1, 1)
            return carry

        lax.fori_loop(0, NCHUNK // 2, body, 0)
        if async_scatter:
            swait(0)
            swait(1)
        plsc.subcore_barrier()

        pltpu.sync_copy(acc.at[pl.ds(sid * RPT, RPT)],
                        out_hbm.at[cid, pl.ds(sid * RPT, RPT)])

    return k(e4, v3, y)


def _pack_edges(adj_edge_index, adj_values):
    row = adj_edge_index[0].astype(jnp.int32)
    col = adj_edge_index[1].astype(jnp.int32)
    e = jnp.stack([row, col])
    pad = jnp.zeros((2, EPAD - N_EDGES), jnp.int32)
    e = jnp.concatenate([e, pad], axis=1)
    v = jnp.concatenate(
        [adj_values, jnp.zeros((EPAD - N_EDGES,), jnp.float32)])
    e4 = e.reshape(2, NW, NCHUNK, CHUNK).transpose(1, 2, 0, 3)
    v3 = v.reshape(NW, NCHUNK, CHUNK)
    return e4, v3


def _unpack_perm(k):
    idx = []
    for f in range(k // 32):
        idx.extend(range(32 * f, 32 * f + 32, 2))
        idx.extend(range(32 * f + 1, 32 * f + 32, 2))
    return jnp.array(idx, jnp.int32)


@jax.jit
def kernel(adj_edge_index, adj_values, x, W1, W2):
    e4, v3 = _pack_edges(adj_edge_index, adj_values)
    w2p = jnp.zeros((W2.shape[0], 48), jnp.float32).at[:, :W2.shape[1]].set(W2)
    w2p = w2p[_unpack_perm(W2.shape[0])]

    y1 = _matmul(x, W1)
    p1 = _spmm_partials(e4, v3, y1, 128, False)
    y2 = _relu_matmul(p1, w2p)
    p2 = _spmm_partials(e4, v3, y2, 48, False, True)
    return _add_slice(p2)
